# Initial kernel scaffold; baseline (speedup 1.0000x reference)
import jax
import jax.numpy as jnp
from jax.experimental import pallas as pl


def kernel(x, edge_index, W0, b0, W1, b1, W2, b2, p1, p2, Wc, bc):
    raise NotImplementedError("write your pallas kernel here")


# speedup vs baseline: 1.1574x; 1.1574x over previous
import math

import jax
import jax.numpy as jnp
from jax.experimental import pallas as pl
from jax.experimental.pallas import tpu as pltpu

_LANE = 128


def _pad_up(n, m):
    return ((n + m - 1) // m) * m


def _colmat(v):
    return v[:, None] * jnp.ones((1, _LANE), jnp.float32)


def _rowscale_mm(x, w, smat):
    m, d = x.shape
    bm = min(512, m)

    def kern(x_ref, w_ref, s_ref, o_ref):
        o_ref[...] = s_ref[...] * jnp.dot(
            x_ref[...], w_ref[...], preferred_element_type=jnp.float32
        )

    return pl.pallas_call(
        kern,
        grid=(m // bm,),
        in_specs=[
            pl.BlockSpec((bm, d), lambda i: (i, 0)),
            pl.BlockSpec((d, w.shape[1]), lambda i: (0, 0)),
            pl.BlockSpec((bm, _LANE), lambda i: (i, 0)),
        ],
        out_specs=pl.BlockSpec((bm, w.shape[1]), lambda i: (i, 0)),
        out_shape=jax.ShapeDtypeStruct((m, w.shape[1]), jnp.float32),
    )(x, w, smat)


def _agg(a, w, dinvmat, bias2d, pmat, transpose):
    m = a.shape[1] if transpose else a.shape[0]
    kdim = a.shape[0] if transpose else a.shape[1]
    d = w.shape[1]
    bm = min(512, m)
    bk = min(1024, kdim)
    jt, it = m // bm, kdim // bk
    nsteps = it

    def kern(a_ref, w_ref, dinv_ref, b_ref, p_ref, wj_ref, h_ref, s_ref):
        i = pl.program_id(1)
        if transpose:
            part = jax.lax.dot_general(
                a_ref[...], w_ref[...],
                dimension_numbers=(((0,), (0,)), ((), ())),
                preferred_element_type=jnp.float32,
            )
        else:
            part = jnp.dot(a_ref[...], w_ref[...], preferred_element_type=jnp.float32)

        @pl.when(i == 0)
        def _():
            h_ref[...] = part

        @pl.when(i > 0)
        def _():
            h_ref[...] += part

        @pl.when(i == nsteps - 1)
        def _():
            acc = h_ref[...]
            if transpose:
                acc = acc + 2.0 * wj_ref[...]
            h = jnp.maximum(dinv_ref[...] * acc + b_ref[...], 0.0)
            h_ref[...] = h
            s_ref[...] = jnp.tanh(
                jnp.dot(h, p_ref[...], preferred_element_type=jnp.float32)
            )

    if transpose:
        a_spec = pl.BlockSpec((bk, bm), lambda j, i: (i, j))
    else:
        a_spec = pl.BlockSpec((bm, bk), lambda j, i: (j, i))

    return pl.pallas_call(
        kern,
        grid=(jt, it),
        in_specs=[
            a_spec,
            pl.BlockSpec((bk, d), lambda j, i: (i, 0)),
            pl.BlockSpec((bm, _LANE), lambda j, i: (j, 0)),
            pl.BlockSpec((1, _LANE), lambda j, i: (0, 0)),
            pl.BlockSpec((d, _LANE), lambda j, i: (0, 0)),
            pl.BlockSpec((bm, d), lambda j, i: (j, 0)),
        ],
        out_specs=[
            pl.BlockSpec((bm, d), lambda j, i: (j, 0)),
            pl.BlockSpec((bm, _LANE), lambda j, i: (j, 0)),
        ],
        out_shape=[
            jax.ShapeDtypeStruct((m, d), jnp.float32),
            jax.ShapeDtypeStruct((m, _LANE), jnp.float32),
        ],
    )(a, w, dinvmat, bias2d, pmat, w)


def _mm_nt_zerodiag(l, r):
    m, kdim = l.shape
    n = r.shape[0]
    bm = min(512, m)
    bn = min(512, n)
    bk = min(1024, kdim)
    kt = kdim // bk

    def kern(l_ref, r_ref, o_ref):
        i, j, k = pl.program_id(0), pl.program_id(1), pl.program_id(2)
        part = jax.lax.dot_general(
            l_ref[...], r_ref[...],
            dimension_numbers=(((1,), (1,)), ((), ())),
            preferred_element_type=jnp.float32,
        )

        @pl.when(k == 0)
        def _():
            o_ref[...] = part

        @pl.when(k > 0)
        def _():
            o_ref[...] += part

        @pl.when(k == kt - 1)
        def _():
            rows = i * bm + jax.lax.broadcasted_iota(jnp.int32, (bm, bn), 0)
            cols = j * bn + jax.lax.broadcasted_iota(jnp.int32, (bm, bn), 1)
            o_ref[...] = jnp.where(rows == cols, 0.0, o_ref[...])

    return pl.pallas_call(
        kern,
        grid=(m // bm, n // bn, kt),
        in_specs=[
            pl.BlockSpec((bm, bk), lambda i, j, k: (i, k)),
            pl.BlockSpec((bn, bk), lambda i, j, k: (j, k)),
        ],
        out_specs=pl.BlockSpec((bm, bn), lambda i, j, k: (i, j)),
        out_shape=jax.ShapeDtypeStruct((m, n), jnp.float32),
    )(l, r)


def _colsum(a):
    m, n = a.shape
    bm = min(1024, m)
    bn = min(512, n)

    def kern(a_ref, o_ref):
        i = pl.program_id(1)
        part = jnp.sum(a_ref[...], axis=0, keepdims=True)

        @pl.when(i == 0)
        def _():
            o_ref[...] = jnp.broadcast_to(part, (8, bn))

        @pl.when(i > 0)
        def _():
            o_ref[...] += jnp.broadcast_to(part, (8, bn))

    return pl.pallas_call(
        kern,
        grid=(n // bn, m // bm),
        in_specs=[pl.BlockSpec((bm, bn), lambda j, i: (i, j))],
        out_specs=pl.BlockSpec((8, bn), lambda j, i: (0, j)),
        out_shape=jax.ShapeDtypeStruct((8, n), jnp.float32),
    )(a)


def _transpose(a):
    m, n = a.shape
    b = min(512, min(m, n))

    def kern(a_ref, o_ref):
        o_ref[...] = a_ref[...].T

    return pl.pallas_call(
        kern,
        grid=(m // b, n // b),
        in_specs=[pl.BlockSpec((b, b), lambda i, j: (i, j))],
        out_specs=pl.BlockSpec((b, b), lambda i, j: (j, i)),
        out_shape=jax.ShapeDtypeStruct((n, m), jnp.float32),
    )(a)


def _gather_rows(srcs, idx, dpos=None):
    r = idx.shape[0]
    n_dense = 2 if dpos is not None else 0

    def kern(*refs):
        if dpos is not None:
            _, dpos_ref = refs[0], refs[1]
            ins = refs[2:2 + len(srcs)]
            outs = refs[2 + len(srcs):]
            i = pl.program_id(0)
            d = dpos_ref[i]
            for t, (s_ref, o_ref) in enumerate(zip(ins, outs)):
                if t < n_dense:
                    lane = jax.lax.broadcasted_iota(
                        jnp.int32, (1, 1, s_ref.shape[2]), 2
                    )
                    o_ref[...] = jnp.where(lane == d, 1.0, s_ref[...])
                else:
                    o_ref[...] = s_ref[...]
        else:
            ins = refs[1:1 + len(srcs)]
            outs = refs[1 + len(srcs):]
            for s_ref, o_ref in zip(ins, outs):
                o_ref[...] = s_ref[...]

    nsp = 1 if dpos is None else 2
    in_specs = []
    out_specs = []
    out_shape = []
    for s in srcs:
        c = s.shape[1]
        in_specs.append(
            pl.BlockSpec((1, 1, c), lambda i, *sref, _c=c: (sref[0][i], 0, 0))
        )
        out_specs.append(pl.BlockSpec((1, 1, c), lambda i, *sref: (i, 0, 0)))
        out_shape.append(jax.ShapeDtypeStruct((r, 1, c), jnp.float32))

    grid_spec = pltpu.PrefetchScalarGridSpec(
        num_scalar_prefetch=nsp,
        grid=(r,),
        in_specs=in_specs,
        out_specs=out_specs,
    )
    args = (idx,) if dpos is None else (idx, dpos)
    outs = pl.pallas_call(kern, grid_spec=grid_spec, out_shape=out_shape)(
        *args, *[s[:, None, :] for s in srcs]
    )
    return [o[:, 0, :] for o in outs]


def _final_head(h2, nreal, wcp, bcp, ncls):
    m, d = h2.shape

    def kern(h_ref, wc_ref, bc_ref, emb_ref, lp_ref):
        rows = jax.lax.broadcasted_iota(jnp.int32, (m, 1), 0)
        h = jnp.where(rows < nreal, h_ref[...], 0.0)
        emb = jnp.sum(h, axis=0, keepdims=True) / float(nreal)
        nrm = jnp.sqrt(jnp.sum(emb * emb))
        emb = emb / jnp.maximum(nrm, 1e-12)
        logits = jnp.dot(emb, wc_ref[...], preferred_element_type=jnp.float32)
        logits = logits + bc_ref[...]
        lanes = jax.lax.broadcasted_iota(jnp.int32, (1, _LANE), 1)
        valid = lanes < ncls
        ml = jnp.where(valid, logits, -jnp.inf)
        mx = jnp.max(ml)
        z = jnp.where(valid, jnp.exp(ml - mx), 0.0)
        lse = mx + jnp.log(jnp.sum(z))
        emb_ref[...] = jnp.broadcast_to(emb, (8, _LANE))
        lp_ref[...] = jnp.broadcast_to(logits - lse, (8, _LANE))

    return pl.pallas_call(
        kern,
        grid=(1,),
        in_specs=[
            pl.BlockSpec((m, d), lambda i: (0, 0)),
            pl.BlockSpec((d, _LANE), lambda i: (0, 0)),
            pl.BlockSpec((1, _LANE), lambda i: (0, 0)),
        ],
        out_specs=[
            pl.BlockSpec((8, _LANE), lambda i: (0, 0)),
            pl.BlockSpec((8, _LANE), lambda i: (0, 0)),
        ],
        out_shape=[
            jax.ShapeDtypeStruct((8, _LANE), jnp.float32),
            jax.ShapeDtypeStruct((8, _LANE), jnp.float32),
        ],
    )(h2, wcp, bcp)


def kernel(x, edge_index, W0, b0, W1, b1, W2, b2, p1, p2, Wc, bc):
    n, d = x.shape
    ncls = Wc.shape[1]
    np_ = _pad_up(n + 1, 512)
    k1 = math.ceil(0.5 * n)
    k1p = _pad_up(k1 + 1, 512)
    k2 = math.ceil(0.5 * k1)
    k2p = _pad_up(k2 + 1, 512)

    r, c = edge_index[0], edge_index[1]
    ii = jnp.arange(n)

    selfw = jnp.where(r == c, 1.0, 0.0)
    diagcnt = jnp.zeros((n,), jnp.float32).at[r].add(selfw)
    newdiag = jnp.where(diagcnt != 0.0, diagcnt, 2.0)

    at = jnp.zeros((np_, np_), jnp.float32).at[c, r].add(1.0)
    at_sl = at.at[ii, ii].set(newdiag)
    bt = at.at[ii, ii].set(1.0)
    bmat = jnp.zeros((np_, np_), jnp.float32).at[r, c].add(1.0).at[ii, ii].set(1.0)

    deg0 = jnp.zeros((n,), jnp.float32).at[c].add(1.0 - selfw) + newdiag
    dinv0 = jnp.where(deg0 > 0, jax.lax.rsqrt(deg0), 0.0)
    dinv0p = jnp.pad(dinv0, (0, np_ - n))

    xp = jnp.pad(x, ((0, np_ - n), (0, 0)))
    b0m, b1m, b2m = b0[None, :], b1[None, :], b2[None, :]
    p1m = _colmat(p1 / jnp.linalg.norm(p1))
    p2m = _colmat(p2 / jnp.linalg.norm(p2))

    w0 = _rowscale_mm(xp, W0, _colmat(dinv0p))
    h0, score1 = _agg(at_sl, w0, _colmat(dinv0p), b0m, p1m, transpose=False)

    vals1, perm1 = jax.lax.top_k(score1[:n, 0], k1)
    order1 = jnp.argsort(perm1)
    perm1 = perm1[order1]
    vals1 = vals1[order1]
    idx1 = jnp.concatenate([perm1, jnp.full((k1p - k1,), np_ - 1, jnp.int32)])
    vals1p = jnp.pad(vals1, (0, k1p - k1))

    br1, btr1, h0g = _gather_rows([bmat, bt, h0], idx1)
    ap1 = _mm_nt_zerodiag(br1, btr1)

    cs1 = _colsum(ap1)[0]
    deg1 = cs1 + 2.0
    dinv1 = jax.lax.rsqrt(deg1)

    w1 = _rowscale_mm(h0g, W1, _colmat(dinv1 * vals1p))
    h1, score2 = _agg(ap1, w1, _colmat(dinv1), b1m, p2m, transpose=True)

    vals2, perm2 = jax.lax.top_k(score2[:k1, 0], k2)
    order2 = jnp.argsort(perm2)
    perm2 = perm2[order2]
    vals2 = vals2[order2]
    idx2 = jnp.concatenate([perm2, jnp.full((k2p - k2,), k1p - 1, jnp.int32)])
    dpos2 = jnp.concatenate([perm2, jnp.full((k2p - k2,), -1, jnp.int32)])
    vals2p = jnp.pad(vals2, (0, k2p - k2))

    ap1t = _transpose(ap1)
    cr2, ctr2, h1g = _gather_rows([ap1, ap1t, h1], idx2, dpos=dpos2)
    ap2 = _mm_nt_zerodiag(cr2, ctr2)

    cs2 = _colsum(ap2)[0]
    deg2 = cs2 + 2.0
    dinv2 = jax.lax.rsqrt(deg2)

    w2 = _rowscale_mm(h1g, W2, _colmat(dinv2 * vals2p))
    h2, _ = _agg(ap2, w2, _colmat(dinv2), b2m, p2m, transpose=True)

    wcp = jnp.zeros((d, _LANE), jnp.float32).at[:, :ncls].set(Wc)
    bcp = jnp.zeros((1, _LANE), jnp.float32).at[0, :ncls].set(bc)
    emb8, lp8 = _final_head(h2, k2, wcp, bcp, ncls)
    return emb8[0:1, :], lp8[0:1, :ncls]


# speedup vs baseline: 1.2098x; 1.0453x over previous
import math

import jax
import jax.numpy as jnp
from jax.experimental import pallas as pl
from jax.experimental.pallas import tpu as pltpu

_LANE = 128


def _pad_up(n, m):
    return ((n + m - 1) // m) * m


def _colmat(v):
    return v[:, None] * jnp.ones((1, _LANE), jnp.float32)


def _rowscale_mm(x, w, smat):
    m, d = x.shape
    bm = min(512, m)

    def kern(x_ref, w_ref, s_ref, o_ref):
        o_ref[...] = s_ref[...] * jnp.dot(
            x_ref[...], w_ref[...], preferred_element_type=jnp.float32
        )

    return pl.pallas_call(
        kern,
        grid=(m // bm,),
        in_specs=[
            pl.BlockSpec((bm, d), lambda i: (i, 0)),
            pl.BlockSpec((d, w.shape[1]), lambda i: (0, 0)),
            pl.BlockSpec((bm, _LANE), lambda i: (i, 0)),
        ],
        out_specs=pl.BlockSpec((bm, w.shape[1]), lambda i: (i, 0)),
        out_shape=jax.ShapeDtypeStruct((m, w.shape[1]), jnp.float32),
    )(x, w, smat)


def _agg(a, w, dinvmat, bias2d, pmat, transpose):
    m = a.shape[1] if transpose else a.shape[0]
    kdim = a.shape[0] if transpose else a.shape[1]
    d = w.shape[1]
    bm = min(512, m)
    bk = min(1024, kdim)
    jt, it = m // bm, kdim // bk
    nsteps = it

    def kern(a_ref, w_ref, dinv_ref, b_ref, p_ref, wj_ref, h_ref, s_ref):
        i = pl.program_id(1)
        if transpose:
            part = jax.lax.dot_general(
                a_ref[...], w_ref[...],
                dimension_numbers=(((0,), (0,)), ((), ())),
                preferred_element_type=jnp.float32,
            )
        else:
            part = jnp.dot(a_ref[...], w_ref[...], preferred_element_type=jnp.float32)

        @pl.when(i == 0)
        def _():
            h_ref[...] = part

        @pl.when(i > 0)
        def _():
            h_ref[...] += part

        @pl.when(i == nsteps - 1)
        def _():
            acc = h_ref[...]
            if transpose:
                acc = acc + 2.0 * wj_ref[...]
            h = jnp.maximum(dinv_ref[...] * acc + b_ref[...], 0.0)
            h_ref[...] = h
            s_ref[...] = jnp.tanh(
                jnp.dot(h, p_ref[...], preferred_element_type=jnp.float32)
            )

    if transpose:
        a_spec = pl.BlockSpec((bk, bm), lambda j, i: (i, j))
    else:
        a_spec = pl.BlockSpec((bm, bk), lambda j, i: (j, i))

    return pl.pallas_call(
        kern,
        grid=(jt, it),
        in_specs=[
            a_spec,
            pl.BlockSpec((bk, d), lambda j, i: (i, 0)),
            pl.BlockSpec((bm, _LANE), lambda j, i: (j, 0)),
            pl.BlockSpec((1, _LANE), lambda j, i: (0, 0)),
            pl.BlockSpec((d, _LANE), lambda j, i: (0, 0)),
            pl.BlockSpec((bm, d), lambda j, i: (j, 0)),
        ],
        out_specs=[
            pl.BlockSpec((bm, d), lambda j, i: (j, 0)),
            pl.BlockSpec((bm, _LANE), lambda j, i: (j, 0)),
        ],
        out_shape=[
            jax.ShapeDtypeStruct((m, d), jnp.float32),
            jax.ShapeDtypeStruct((m, _LANE), jnp.float32),
        ],
    )(a, w, dinvmat, bias2d, pmat, w)


def _mm_nt_zerodiag(l, r, with_transpose=False):
    m, kdim = l.shape
    n = r.shape[0]
    bm = min(512, m)
    bn = min(512, n)
    bk = min(1024, kdim)
    kt = kdim // bk

    def kern(l_ref, r_ref, o_ref, *maybe_t):
        i, j, k = pl.program_id(0), pl.program_id(1), pl.program_id(2)
        part = jax.lax.dot_general(
            l_ref[...], r_ref[...],
            dimension_numbers=(((1,), (1,)), ((), ())),
            preferred_element_type=jnp.float32,
        )

        @pl.when(k == 0)
        def _():
            o_ref[...] = part

        @pl.when(k > 0)
        def _():
            o_ref[...] += part

        @pl.when(k == kt - 1)
        def _():
            rows = i * bm + jax.lax.broadcasted_iota(jnp.int32, (bm, bn), 0)
            cols = j * bn + jax.lax.broadcasted_iota(jnp.int32, (bm, bn), 1)
            res = jnp.where(rows == cols, 0.0, o_ref[...])
            o_ref[...] = res
            if with_transpose:
                maybe_t[0][...] = res.T

    out_specs = [pl.BlockSpec((bm, bn), lambda i, j, k: (i, j))]
    out_shape = [jax.ShapeDtypeStruct((m, n), jnp.float32)]
    if with_transpose:
        out_specs.append(pl.BlockSpec((bn, bm), lambda i, j, k: (j, i)))
        out_shape.append(jax.ShapeDtypeStruct((n, m), jnp.float32))

    return pl.pallas_call(
        kern,
        grid=(m // bm, n // bn, kt),
        in_specs=[
            pl.BlockSpec((bm, bk), lambda i, j, k: (i, k)),
            pl.BlockSpec((bn, bk), lambda i, j, k: (j, k)),
        ],
        out_specs=out_specs,
        out_shape=out_shape,
    )(l, r)


def _colsum(a):
    m, n = a.shape
    bm = min(1024, m)
    bn = min(512, n)

    def kern(a_ref, o_ref):
        i = pl.program_id(1)
        part = jnp.sum(a_ref[...], axis=0, keepdims=True)

        @pl.when(i == 0)
        def _():
            o_ref[...] = jnp.broadcast_to(part, (8, bn))

        @pl.when(i > 0)
        def _():
            o_ref[...] += jnp.broadcast_to(part, (8, bn))

    return pl.pallas_call(
        kern,
        grid=(n // bn, m // bm),
        in_specs=[pl.BlockSpec((bm, bn), lambda j, i: (i, j))],
        out_specs=pl.BlockSpec((8, bn), lambda j, i: (0, j)),
        out_shape=jax.ShapeDtypeStruct((8, n), jnp.float32),
    )(a)


def _gather_rows(srcs, idx, dpos=None):
    r = idx.shape[0]
    n_dense = 2 if dpos is not None else 0

    def kern(*refs):
        if dpos is not None:
            _, dpos_ref = refs[0], refs[1]
            ins = refs[2:2 + len(srcs)]
            outs = refs[2 + len(srcs):]
            i = pl.program_id(0)
            d = dpos_ref[i]
            for t, (s_ref, o_ref) in enumerate(zip(ins, outs)):
                if t < n_dense:
                    lane = jax.lax.broadcasted_iota(
                        jnp.int32, (1, 1, s_ref.shape[2]), 2
                    )
                    o_ref[...] = jnp.where(lane == d, 1.0, s_ref[...])
                else:
                    o_ref[...] = s_ref[...]
        else:
            ins = refs[1:1 + len(srcs)]
            outs = refs[1 + len(srcs):]
            for s_ref, o_ref in zip(ins, outs):
                o_ref[...] = s_ref[...]

    nsp = 1 if dpos is None else 2
    in_specs = []
    out_specs = []
    out_shape = []
    for s in srcs:
        c = s.shape[1]
        in_specs.append(
            pl.BlockSpec((1, 1, c), lambda i, *sref, _c=c: (sref[0][i], 0, 0))
        )
        out_specs.append(pl.BlockSpec((1, 1, c), lambda i, *sref: (i, 0, 0)))
        out_shape.append(jax.ShapeDtypeStruct((r, 1, c), jnp.float32))

    grid_spec = pltpu.PrefetchScalarGridSpec(
        num_scalar_prefetch=nsp,
        grid=(r,),
        in_specs=in_specs,
        out_specs=out_specs,
    )
    args = (idx,) if dpos is None else (idx, dpos)
    outs = pl.pallas_call(kern, grid_spec=grid_spec, out_shape=out_shape)(
        *args, *[s[:, None, :] for s in srcs]
    )
    return [o[:, 0, :] for o in outs]


def _final_head(h2, nreal, wcp, bcp, ncls):
    m, d = h2.shape

    def kern(h_ref, wc_ref, bc_ref, emb_ref, lp_ref):
        rows = jax.lax.broadcasted_iota(jnp.int32, (m, 1), 0)
        h = jnp.where(rows < nreal, h_ref[...], 0.0)
        emb = jnp.sum(h, axis=0, keepdims=True) / float(nreal)
        nrm = jnp.sqrt(jnp.sum(emb * emb))
        emb = emb / jnp.maximum(nrm, 1e-12)
        logits = jnp.dot(emb, wc_ref[...], preferred_element_type=jnp.float32)
        logits = logits + bc_ref[...]
        lanes = jax.lax.broadcasted_iota(jnp.int32, (1, _LANE), 1)
        valid = lanes < ncls
        ml = jnp.where(valid, logits, -jnp.inf)
        mx = jnp.max(ml)
        z = jnp.where(valid, jnp.exp(ml - mx), 0.0)
        lse = mx + jnp.log(jnp.sum(z))
        emb_ref[...] = jnp.broadcast_to(emb, (8, _LANE))
        lp_ref[...] = jnp.broadcast_to(logits - lse, (8, _LANE))

    return pl.pallas_call(
        kern,
        grid=(1,),
        in_specs=[
            pl.BlockSpec((m, d), lambda i: (0, 0)),
            pl.BlockSpec((d, _LANE), lambda i: (0, 0)),
            pl.BlockSpec((1, _LANE), lambda i: (0, 0)),
        ],
        out_specs=[
            pl.BlockSpec((8, _LANE), lambda i: (0, 0)),
            pl.BlockSpec((8, _LANE), lambda i: (0, 0)),
        ],
        out_shape=[
            jax.ShapeDtypeStruct((8, _LANE), jnp.float32),
            jax.ShapeDtypeStruct((8, _LANE), jnp.float32),
        ],
    )(h2, wcp, bcp)


def kernel(x, edge_index, W0, b0, W1, b1, W2, b2, p1, p2, Wc, bc):
    n, d = x.shape
    ncls = Wc.shape[1]
    np_ = _pad_up(n + 1, 512)
    k1 = math.ceil(0.5 * n)
    k1p = _pad_up(k1 + 1, 512)
    k2 = math.ceil(0.5 * k1)
    k2p = _pad_up(k2 + 1, 512)

    r, c = edge_index[0], edge_index[1]
    ii = jnp.arange(n)

    selfw = jnp.where(r == c, 1.0, 0.0)
    diagcnt = jnp.zeros((n,), jnp.float32).at[r].add(selfw)
    newdiag = jnp.where(diagcnt != 0.0, diagcnt, 2.0)

    at_sl = (
        jnp.zeros((np_, np_), jnp.float32)
        .at[jnp.concatenate([c, ii]), jnp.concatenate([r, ii])]
        .add(jnp.concatenate([jnp.ones_like(selfw), newdiag - diagcnt]))
    )

    deg0 = jnp.zeros((n,), jnp.float32).at[c].add(1.0 - selfw) + newdiag
    dinv0 = jnp.where(deg0 > 0, jax.lax.rsqrt(deg0), 0.0)
    dinv0p = jnp.pad(dinv0, (0, np_ - n))

    xp = jnp.pad(x, ((0, np_ - n), (0, 0)))
    b0m, b1m, b2m = b0[None, :], b1[None, :], b2[None, :]
    p1m = _colmat(p1 / jnp.linalg.norm(p1))
    p2m = _colmat(p2 / jnp.linalg.norm(p2))

    w0 = _rowscale_mm(xp, W0, _colmat(dinv0p))
    h0, score1 = _agg(at_sl, w0, _colmat(dinv0p), b0m, p1m, transpose=False)

    vals1, perm1 = jax.lax.top_k(score1[:n, 0], k1)
    order1 = jnp.argsort(perm1)
    perm1 = perm1[order1]
    vals1 = vals1[order1]
    idx1 = jnp.concatenate([perm1, jnp.full((k1p - k1,), np_ - 1, jnp.int32)])
    vals1p = jnp.pad(vals1, (0, k1p - k1))

    g1 = jnp.full((n,), -1, jnp.int32).at[perm1].set(jnp.arange(k1, dtype=jnp.int32))
    ar1 = jnp.arange(k1)
    gr = g1[r]
    br1 = (
        jnp.zeros((k1p, np_), jnp.float32)
        .at[jnp.where(gr >= 0, gr, k1p - 1), c].add(1.0)
        .at[k1p - 1, :].set(0.0)
        .at[ar1, perm1].set(1.0)
    )
    gc = g1[c]
    btr1 = (
        jnp.zeros((k1p, np_), jnp.float32)
        .at[jnp.where(gc >= 0, gc, k1p - 1), r].add(1.0)
        .at[k1p - 1, :].set(0.0)
        .at[ar1, perm1].set(1.0)
    )
    (h0g,) = _gather_rows([h0], idx1)
    ap1, ap1t = _mm_nt_zerodiag(br1, btr1, with_transpose=True)

    cs1 = _colsum(ap1)[0]
    deg1 = cs1 + 2.0
    dinv1 = jax.lax.rsqrt(deg1)

    w1 = _rowscale_mm(h0g, W1, _colmat(dinv1 * vals1p))
    h1, score2 = _agg(ap1, w1, _colmat(dinv1), b1m, p2m, transpose=True)

    vals2, perm2 = jax.lax.top_k(score2[:k1, 0], k2)
    order2 = jnp.argsort(perm2)
    perm2 = perm2[order2]
    vals2 = vals2[order2]
    idx2 = jnp.concatenate([perm2, jnp.full((k2p - k2,), k1p - 1, jnp.int32)])
    dpos2 = jnp.concatenate([perm2, jnp.full((k2p - k2,), -1, jnp.int32)])
    vals2p = jnp.pad(vals2, (0, k2p - k2))

    cr2, ctr2, h1g = _gather_rows([ap1, ap1t, h1], idx2, dpos=dpos2)
    (ap2,) = _mm_nt_zerodiag(cr2, ctr2)

    cs2 = _colsum(ap2)[0]
    deg2 = cs2 + 2.0
    dinv2 = jax.lax.rsqrt(deg2)

    w2 = _rowscale_mm(h1g, W2, _colmat(dinv2 * vals2p))
    h2, _ = _agg(ap2, w2, _colmat(dinv2), b2m, p2m, transpose=True)

    wcp = jnp.zeros((d, _LANE), jnp.float32).at[:, :ncls].set(Wc)
    bcp = jnp.zeros((1, _LANE), jnp.float32).at[0, :ncls].set(bc)
    emb8, lp8 = _final_head(h2, k2, wcp, bcp, ncls)
    return emb8[0:1, :], lp8[0:1, :ncls]


# speedup vs baseline: 1.2751x; 1.0540x over previous
import math

import jax
import jax.numpy as jnp
from jax.experimental import pallas as pl
from jax.experimental.pallas import tpu as pltpu

_LANE = 128


def _pad_up(n, m):
    return ((n + m - 1) // m) * m


def _colmat(v):
    return v[:, None] * jnp.ones((1, _LANE), jnp.float32)


def _rowscale_mm(x, w, smat):
    m, d = x.shape
    bm = min(512, m)

    def kern(x_ref, w_ref, s_ref, o_ref):
        o_ref[...] = s_ref[...] * jnp.dot(
            x_ref[...], w_ref[...], preferred_element_type=jnp.float32
        )

    return pl.pallas_call(
        kern,
        grid=(m // bm,),
        in_specs=[
            pl.BlockSpec((bm, d), lambda i: (i, 0)),
            pl.BlockSpec((d, w.shape[1]), lambda i: (0, 0)),
            pl.BlockSpec((bm, _LANE), lambda i: (i, 0)),
        ],
        out_specs=pl.BlockSpec((bm, w.shape[1]), lambda i: (i, 0)),
        out_shape=jax.ShapeDtypeStruct((m, w.shape[1]), jnp.float32),
    )(x, w, smat)


def _agg(a, w, dinvmat, bias2d, pmat, transpose):
    m = a.shape[1] if transpose else a.shape[0]
    kdim = a.shape[0] if transpose else a.shape[1]
    d = w.shape[1]
    bm = min(512, m)
    bk = min(1024, kdim)
    jt, it = m // bm, kdim // bk
    nsteps = it

    def kern(a_ref, w_ref, dinv_ref, b_ref, p_ref, wj_ref, h_ref, s_ref):
        i = pl.program_id(1)
        if transpose:
            part = jax.lax.dot_general(
                a_ref[...], w_ref[...],
                dimension_numbers=(((0,), (0,)), ((), ())),
                preferred_element_type=jnp.float32,
            )
        else:
            part = jnp.dot(a_ref[...], w_ref[...], preferred_element_type=jnp.float32)

        @pl.when(i == 0)
        def _():
            h_ref[...] = part

        @pl.when(i > 0)
        def _():
            h_ref[...] += part

        @pl.when(i == nsteps - 1)
        def _():
            acc = h_ref[...]
            if transpose:
                acc = acc + 2.0 * wj_ref[...]
            h = jnp.maximum(dinv_ref[...] * acc + b_ref[...], 0.0)
            h_ref[...] = h
            s_ref[...] = jnp.tanh(
                jnp.dot(h, p_ref[...], preferred_element_type=jnp.float32)
            )

    if transpose:
        a_spec = pl.BlockSpec((bk, bm), lambda j, i: (i, j))
    else:
        a_spec = pl.BlockSpec((bm, bk), lambda j, i: (j, i))

    return pl.pallas_call(
        kern,
        grid=(jt, it),
        in_specs=[
            a_spec,
            pl.BlockSpec((bk, d), lambda j, i: (i, 0)),
            pl.BlockSpec((bm, _LANE), lambda j, i: (j, 0)),
            pl.BlockSpec((1, _LANE), lambda j, i: (0, 0)),
            pl.BlockSpec((d, _LANE), lambda j, i: (0, 0)),
            pl.BlockSpec((bm, d), lambda j, i: (j, 0)),
        ],
        out_specs=[
            pl.BlockSpec((bm, d), lambda j, i: (j, 0)),
            pl.BlockSpec((bm, _LANE), lambda j, i: (j, 0)),
        ],
        out_shape=[
            jax.ShapeDtypeStruct((m, d), jnp.float32),
            jax.ShapeDtypeStruct((m, _LANE), jnp.float32),
        ],
    )(a, w, dinvmat, bias2d, pmat, w)


def _mm_nt_zerodiag(l, r, with_transpose=False):
    m, kdim = l.shape
    n = r.shape[0]
    bm = 1024 if m % 1024 == 0 else min(512, m)
    bn = 1024 if n % 1024 == 0 else min(512, n)
    bk = min(512, kdim)
    kt = kdim // bk

    def kern(l_ref, r_ref, o_ref, *maybe_t):
        i, j, k = pl.program_id(0), pl.program_id(1), pl.program_id(2)
        part = jax.lax.dot_general(
            l_ref[...], r_ref[...],
            dimension_numbers=(((1,), (1,)), ((), ())),
            preferred_element_type=jnp.float32,
        )

        @pl.when(k == 0)
        def _():
            o_ref[...] = part

        @pl.when(k > 0)
        def _():
            o_ref[...] += part

        @pl.when(k == kt - 1)
        def _():
            rows = i * bm + jax.lax.broadcasted_iota(jnp.int32, (bm, bn), 0)
            cols = j * bn + jax.lax.broadcasted_iota(jnp.int32, (bm, bn), 1)
            res = jnp.where(rows == cols, 0.0, o_ref[...])
            o_ref[...] = res
            if with_transpose:
                maybe_t[0][...] = res.T

    out_specs = [pl.BlockSpec((bm, bn), lambda i, j, k: (i, j))]
    out_shape = [jax.ShapeDtypeStruct((m, n), jnp.float32)]
    if with_transpose:
        out_specs.append(pl.BlockSpec((bn, bm), lambda i, j, k: (j, i)))
        out_shape.append(jax.ShapeDtypeStruct((n, m), jnp.float32))

    return pl.pallas_call(
        kern,
        grid=(m // bm, n // bn, kt),
        in_specs=[
            pl.BlockSpec((bm, bk), lambda i, j, k: (i, k)),
            pl.BlockSpec((bn, bk), lambda i, j, k: (j, k)),
        ],
        out_specs=out_specs,
        out_shape=out_shape,
    )(l, r)


def _colsum(a):
    m, n = a.shape
    bm = min(1024, m)
    bn = min(512, n)

    def kern(a_ref, o_ref):
        i = pl.program_id(1)
        part = jnp.sum(a_ref[...], axis=0, keepdims=True)

        @pl.when(i == 0)
        def _():
            o_ref[...] = jnp.broadcast_to(part, (8, bn))

        @pl.when(i > 0)
        def _():
            o_ref[...] += jnp.broadcast_to(part, (8, bn))

    return pl.pallas_call(
        kern,
        grid=(n // bn, m // bm),
        in_specs=[pl.BlockSpec((bm, bn), lambda j, i: (i, j))],
        out_specs=pl.BlockSpec((8, bn), lambda j, i: (0, j)),
        out_shape=jax.ShapeDtypeStruct((8, n), jnp.float32),
    )(a)


def _gather_rows(srcs, idx, dpos=None):
    r = idx.shape[0]
    n_dense = 2 if dpos is not None else 0

    def kern(*refs):
        if dpos is not None:
            _, dpos_ref = refs[0], refs[1]
            ins = refs[2:2 + len(srcs)]
            outs = refs[2 + len(srcs):]
            i = pl.program_id(0)
            d = dpos_ref[i]
            for t, (s_ref, o_ref) in enumerate(zip(ins, outs)):
                if t < n_dense:
                    lane = jax.lax.broadcasted_iota(
                        jnp.int32, (1, 1, s_ref.shape[2]), 2
                    )
                    o_ref[...] = jnp.where(lane == d, 1.0, s_ref[...])
                else:
                    o_ref[...] = s_ref[...]
        else:
            ins = refs[1:1 + len(srcs)]
            outs = refs[1 + len(srcs):]
            for s_ref, o_ref in zip(ins, outs):
                o_ref[...] = s_ref[...]

    nsp = 1 if dpos is None else 2
    in_specs = []
    out_specs = []
    out_shape = []
    for s in srcs:
        c = s.shape[1]
        in_specs.append(
            pl.BlockSpec((1, 1, c), lambda i, *sref, _c=c: (sref[0][i], 0, 0))
        )
        out_specs.append(pl.BlockSpec((1, 1, c), lambda i, *sref: (i, 0, 0)))
        out_shape.append(jax.ShapeDtypeStruct((r, 1, c), jnp.float32))

    grid_spec = pltpu.PrefetchScalarGridSpec(
        num_scalar_prefetch=nsp,
        grid=(r,),
        in_specs=in_specs,
        out_specs=out_specs,
    )
    args = (idx,) if dpos is None else (idx, dpos)
    outs = pl.pallas_call(kern, grid_spec=grid_spec, out_shape=out_shape)(
        *args, *[s[:, None, :] for s in srcs]
    )
    return [o[:, 0, :] for o in outs]


def _final_head(h2, nreal, wcp, bcp, ncls):
    m, d = h2.shape

    def kern(h_ref, wc_ref, bc_ref, emb_ref, lp_ref):
        rows = jax.lax.broadcasted_iota(jnp.int32, (m, 1), 0)
        h = jnp.where(rows < nreal, h_ref[...], 0.0)
        emb = jnp.sum(h, axis=0, keepdims=True) / float(nreal)
        nrm = jnp.sqrt(jnp.sum(emb * emb))
        emb = emb / jnp.maximum(nrm, 1e-12)
        logits = jnp.dot(emb, wc_ref[...], preferred_element_type=jnp.float32)
        logits = logits + bc_ref[...]
        lanes = jax.lax.broadcasted_iota(jnp.int32, (1, _LANE), 1)
        valid = lanes < ncls
        ml = jnp.where(valid, logits, -jnp.inf)
        mx = jnp.max(ml)
        z = jnp.where(valid, jnp.exp(ml - mx), 0.0)
        lse = mx + jnp.log(jnp.sum(z))
        emb_ref[...] = jnp.broadcast_to(emb, (8, _LANE))
        lp_ref[...] = jnp.broadcast_to(logits - lse, (8, _LANE))

    return pl.pallas_call(
        kern,
        grid=(1,),
        in_specs=[
            pl.BlockSpec((m, d), lambda i: (0, 0)),
            pl.BlockSpec((d, _LANE), lambda i: (0, 0)),
            pl.BlockSpec((1, _LANE), lambda i: (0, 0)),
        ],
        out_specs=[
            pl.BlockSpec((8, _LANE), lambda i: (0, 0)),
            pl.BlockSpec((8, _LANE), lambda i: (0, 0)),
        ],
        out_shape=[
            jax.ShapeDtypeStruct((8, _LANE), jnp.float32),
            jax.ShapeDtypeStruct((8, _LANE), jnp.float32),
        ],
    )(h2, wcp, bcp)


def kernel(x, edge_index, W0, b0, W1, b1, W2, b2, p1, p2, Wc, bc):
    n, d = x.shape
    ncls = Wc.shape[1]
    np_ = _pad_up(n + 1, 512)
    k1 = math.ceil(0.5 * n)
    k1p = _pad_up(k1 + 1, 512)
    k2 = math.ceil(0.5 * k1)
    k2p = _pad_up(k2 + 1, 512)

    r, c = edge_index[0], edge_index[1]
    ii = jnp.arange(n)

    selfw = jnp.where(r == c, 1.0, 0.0)
    diagcnt = jnp.zeros((n,), jnp.float32).at[r].add(selfw)
    newdiag = jnp.where(diagcnt != 0.0, diagcnt, 2.0)

    at_sl = (
        jnp.zeros((np_, np_), jnp.float32)
        .at[jnp.concatenate([c, ii]), jnp.concatenate([r, ii])]
        .add(jnp.concatenate([jnp.ones_like(selfw), newdiag - diagcnt]))
    )

    deg0 = jnp.zeros((n,), jnp.float32).at[c].add(1.0 - selfw) + newdiag
    dinv0 = jnp.where(deg0 > 0, jax.lax.rsqrt(deg0), 0.0)
    dinv0p = jnp.pad(dinv0, (0, np_ - n))

    xp = jnp.pad(x, ((0, np_ - n), (0, 0)))
    b0m, b1m, b2m = b0[None, :], b1[None, :], b2[None, :]
    p1m = _colmat(p1 / jnp.linalg.norm(p1))
    p2m = _colmat(p2 / jnp.linalg.norm(p2))

    w0 = _rowscale_mm(xp, W0, _colmat(dinv0p))
    h0, score1 = _agg(at_sl, w0, _colmat(dinv0p), b0m, p1m, transpose=False)

    vals1, perm1 = jax.lax.top_k(score1[:n, 0], k1)
    order1 = jnp.argsort(perm1)
    perm1 = perm1[order1]
    vals1 = vals1[order1]
    idx1 = jnp.concatenate([perm1, jnp.full((k1p - k1,), np_ - 1, jnp.int32)])
    vals1p = jnp.pad(vals1, (0, k1p - k1))

    g1 = jnp.full((n,), -1, jnp.int32).at[perm1].set(jnp.arange(k1, dtype=jnp.int32))
    ar1 = jnp.arange(k1)
    gr = g1[r]
    br1 = (
        jnp.zeros((k1p, np_), jnp.float32)
        .at[jnp.where(gr >= 0, gr, k1p - 1), c].add(1.0)
        .at[k1p - 1, :].set(0.0)
        .at[ar1, perm1].set(1.0)
    )
    gc = g1[c]
    btr1 = (
        jnp.zeros((k1p, np_), jnp.float32)
        .at[jnp.where(gc >= 0, gc, k1p - 1), r].add(1.0)
        .at[k1p - 1, :].set(0.0)
        .at[ar1, perm1].set(1.0)
    )
    (h0g,) = _gather_rows([h0], idx1)
    ap1, ap1t = _mm_nt_zerodiag(
        br1.astype(jnp.bfloat16), btr1.astype(jnp.bfloat16), with_transpose=True
    )

    cs1 = _colsum(ap1)[0]
    deg1 = cs1 + 2.0
    dinv1 = jax.lax.rsqrt(deg1)

    w1 = _rowscale_mm(h0g, W1, _colmat(dinv1 * vals1p))
    h1, score2 = _agg(ap1, w1, _colmat(dinv1), b1m, p2m, transpose=True)

    vals2, perm2 = jax.lax.top_k(score2[:k1, 0], k2)
    order2 = jnp.argsort(perm2)
    perm2 = perm2[order2]
    vals2 = vals2[order2]
    idx2 = jnp.concatenate([perm2, jnp.full((k2p - k2,), k1p - 1, jnp.int32)])
    dpos2 = jnp.concatenate([perm2, jnp.full((k2p - k2,), -1, jnp.int32)])
    vals2p = jnp.pad(vals2, (0, k2p - k2))

    cr2, ctr2, h1g = _gather_rows([ap1, ap1t, h1], idx2, dpos=dpos2)
    (ap2,) = _mm_nt_zerodiag(cr2.astype(jnp.bfloat16), ctr2.astype(jnp.bfloat16))

    cs2 = _colsum(ap2)[0]
    deg2 = cs2 + 2.0
    dinv2 = jax.lax.rsqrt(deg2)

    w2 = _rowscale_mm(h1g, W2, _colmat(dinv2 * vals2p))
    h2, _ = _agg(ap2, w2, _colmat(dinv2), b2m, p2m, transpose=True)

    wcp = jnp.zeros((d, _LANE), jnp.float32).at[:, :ncls].set(Wc)
    bcp = jnp.zeros((1, _LANE), jnp.float32).at[0, :ncls].set(bc)
    emb8, lp8 = _final_head(h2, k2, wcp, bcp, ncls)
    return emb8[0:1, :], lp8[0:1, :ncls]
